# baseline (speedup 1.0000x reference)
import jax
import jax.numpy as jnp
from jax.experimental import pallas as pl
from jax.experimental.pallas import tpu as pltpu


def _mlp_t_kernel(x_ref, xh_ref, w1_ref, w2_ref, w3_ref, w4_ref,
                  b1_ref, b2_ref, b3_ref, b4_ref, o_ref):
    xall = jnp.concatenate([x_ref[...], xh_ref[...]], axis=0)
    t = jnp.tanh(jnp.dot(w1_ref[...], xall,
                         preferred_element_type=jnp.float32) + b1_ref[...])
    t = jnp.tanh(jnp.dot(w2_ref[...], t,
                         preferred_element_type=jnp.float32) + b2_ref[...])
    t = jnp.tanh(jnp.dot(w3_ref[...], t,
                         preferred_element_type=jnp.float32) + b3_ref[...])
    t = jnp.dot(w4_ref[...], t, preferred_element_type=jnp.float32)
    y = 0.5 * jnp.tanh(t[0:1, :] + b4_ref[0:1, :]) + 0.5
    o_ref[...] = y.astype(o_ref.dtype)


def kernel(x, x_hat, w1, b1, w2, b2, w3, b3, w4, b4, *, batch_tile=32768):
    bs = x.shape[0]
    feat = x.shape[1] * x.shape[2]

    xf = x.transpose(1, 2, 0).reshape(feat, bs).astype(jnp.float32)
    xhf = x_hat.transpose(1, 2, 0).reshape(feat, bs).astype(jnp.float32)

    w1t = 0.5 * w1.astype(jnp.float32).T
    b1c = 0.5 * b1.astype(jnp.float32).reshape(-1, 1)

    def fold(w, b):
        wt = w.astype(jnp.float32).T
        bc = b.astype(jnp.float32).reshape(-1, 1)
        return 0.25 * wt, 0.5 * (0.5 * wt.sum(axis=1, keepdims=True) + bc)

    w2t, b2c = fold(w2, b2)
    w3t, b3c = fold(w3, b3)
    w4f, b4f = fold(w4, b4)
    w4t = jnp.zeros((8, 16), jnp.float32).at[0:1, :].set(w4f)
    b4c = jnp.zeros((8, 1), jnp.float32).at[0:1, :].set(b4f)

    bt = min(batch_tile, bs)
    pad = (-bs) % bt
    if pad:
        xf = jnp.pad(xf, ((0, 0), (0, pad)))
        xhf = jnp.pad(xhf, ((0, 0), (0, pad)))
    bs_p = bs + pad
    grid = bs_p // bt

    out = pl.pallas_call(
        _mlp_t_kernel,
        out_shape=jax.ShapeDtypeStruct((1, bs_p), jnp.float32),
        grid=(grid,),
        in_specs=[
            pl.BlockSpec((feat, bt), lambda i: (0, i)),
            pl.BlockSpec((feat, bt), lambda i: (0, i)),
            pl.BlockSpec(w1t.shape, lambda i: (0, 0)),
            pl.BlockSpec(w2t.shape, lambda i: (0, 0)),
            pl.BlockSpec(w3t.shape, lambda i: (0, 0)),
            pl.BlockSpec(w4t.shape, lambda i: (0, 0)),
            pl.BlockSpec(b1c.shape, lambda i: (0, 0)),
            pl.BlockSpec(b2c.shape, lambda i: (0, 0)),
            pl.BlockSpec(b3c.shape, lambda i: (0, 0)),
            pl.BlockSpec(b4c.shape, lambda i: (0, 0)),
        ],
        out_specs=pl.BlockSpec((1, bt), lambda i: (0, i)),
        compiler_params=pltpu.CompilerParams(
            dimension_semantics=("parallel",)),
    )(xf, xhf, w1t, w2t, w3t, w4t, b1c, b2c, b3c, b4c)

    return out[0, :bs].reshape(bs, 1)


# speedup vs baseline: 1.0012x; 1.0012x over previous
import jax
import jax.numpy as jnp
from jax.experimental import pallas as pl
from jax.experimental.pallas import tpu as pltpu


def _mlp_t_kernel(x_ref, xh_ref, w1_ref, w2_ref, w3_ref, w4_ref,
                  b1_ref, b2_ref, b3_ref, b4_ref, o_ref):
    xall = jnp.concatenate([x_ref[...], xh_ref[...]], axis=0)
    t = jnp.tanh(jnp.dot(w1_ref[...], xall,
                         preferred_element_type=jnp.float32) + b1_ref[...])
    t = jnp.tanh(jnp.dot(w2_ref[...], t,
                         preferred_element_type=jnp.float32) + b2_ref[...])
    t = jnp.tanh(jnp.dot(w3_ref[...], t,
                         preferred_element_type=jnp.float32) + b3_ref[...])
    t = jnp.dot(w4_ref[...], t, preferred_element_type=jnp.float32)
    y = 0.5 * jnp.tanh(t[0:1, :] + b4_ref[0:1, :]) + 0.5
    o_ref[...] = y.astype(o_ref.dtype)


def kernel(x, x_hat, w1, b1, w2, b2, w3, b3, w4, b4, *, batch_tile=16384):
    bs = x.shape[0]
    feat = x.shape[1] * x.shape[2]

    xf = x.transpose(1, 2, 0).reshape(feat, bs).astype(jnp.float32)
    xhf = x_hat.transpose(1, 2, 0).reshape(feat, bs).astype(jnp.float32)

    w1t = 0.5 * w1.astype(jnp.float32).T
    b1c = 0.5 * b1.astype(jnp.float32).reshape(-1, 1)

    def fold(w, b):
        wt = w.astype(jnp.float32).T
        bc = b.astype(jnp.float32).reshape(-1, 1)
        return 0.25 * wt, 0.5 * (0.5 * wt.sum(axis=1, keepdims=True) + bc)

    w2t, b2c = fold(w2, b2)
    w3t, b3c = fold(w3, b3)
    w4f, b4f = fold(w4, b4)
    w4t = jnp.zeros((8, 16), jnp.float32).at[0:1, :].set(w4f)
    b4c = jnp.zeros((8, 1), jnp.float32).at[0:1, :].set(b4f)

    bt = min(batch_tile, bs)
    pad = (-bs) % bt
    if pad:
        xf = jnp.pad(xf, ((0, 0), (0, pad)))
        xhf = jnp.pad(xhf, ((0, 0), (0, pad)))
    bs_p = bs + pad
    grid = bs_p // bt

    out = pl.pallas_call(
        _mlp_t_kernel,
        out_shape=jax.ShapeDtypeStruct((1, bs_p), jnp.float32),
        grid=(grid,),
        in_specs=[
            pl.BlockSpec((feat, bt), lambda i: (0, i)),
            pl.BlockSpec((feat, bt), lambda i: (0, i)),
            pl.BlockSpec(w1t.shape, lambda i: (0, 0)),
            pl.BlockSpec(w2t.shape, lambda i: (0, 0)),
            pl.BlockSpec(w3t.shape, lambda i: (0, 0)),
            pl.BlockSpec(w4t.shape, lambda i: (0, 0)),
            pl.BlockSpec(b1c.shape, lambda i: (0, 0)),
            pl.BlockSpec(b2c.shape, lambda i: (0, 0)),
            pl.BlockSpec(b3c.shape, lambda i: (0, 0)),
            pl.BlockSpec(b4c.shape, lambda i: (0, 0)),
        ],
        out_specs=pl.BlockSpec((1, bt), lambda i: (0, i)),
        compiler_params=pltpu.CompilerParams(
            dimension_semantics=("parallel",)),
    )(xf, xhf, w1t, w2t, w3t, w4t, b1c, b2c, b3c, b4c)

    return out[0, :bs].reshape(bs, 1)
